# Initial kernel scaffold; baseline (speedup 1.0000x reference)
import jax
import jax.numpy as jnp
from jax.experimental import pallas as pl


def kernel(x, emb):
    raise NotImplementedError("write your pallas kernel here")


# speedup vs baseline: 1.6957x; 1.6957x over previous
import jax
import jax.numpy as jnp
from jax.experimental import pallas as pl


S_BLK = 512


def _add_kernel(x_ref, emb_ref, out_ref):
    out_ref[...] = x_ref[...] + emb_ref[...]


def kernel(x, emb):
    batch, seq_len, d_model = x.shape
    n_s = seq_len // S_BLK
    return pl.pallas_call(
        _add_kernel,
        grid=(n_s, batch),
        in_specs=[
            pl.BlockSpec((1, S_BLK, d_model), lambda i_s, i_b: (i_b, i_s, 0)),
            pl.BlockSpec((S_BLK, d_model), lambda i_s, i_b: (i_s, 0)),
        ],
        out_specs=pl.BlockSpec((1, S_BLK, d_model), lambda i_s, i_b: (i_b, i_s, 0)),
        out_shape=jax.ShapeDtypeStruct(x.shape, x.dtype),
    )(x, emb)


# speedup vs baseline: 1.8734x; 1.1048x over previous
import jax
import jax.numpy as jnp
from jax.experimental import pallas as pl


S_BLK = 1024


def _add_kernel(x_ref, emb_ref, out_ref):
    out_ref[...] = x_ref[...] + emb_ref[...]


def kernel(x, emb):
    batch, seq_len, d_model = x.shape
    n_s = seq_len // S_BLK
    return pl.pallas_call(
        _add_kernel,
        grid=(n_s, batch),
        in_specs=[
            pl.BlockSpec((1, S_BLK, d_model), lambda i_s, i_b: (i_b, i_s, 0)),
            pl.BlockSpec((S_BLK, d_model), lambda i_s, i_b: (i_s, 0)),
        ],
        out_specs=pl.BlockSpec((1, S_BLK, d_model), lambda i_s, i_b: (i_b, i_s, 0)),
        out_shape=jax.ShapeDtypeStruct(x.shape, x.dtype),
    )(x, emb)


# speedup vs baseline: 1.9955x; 1.0652x over previous
import jax
import jax.numpy as jnp
from jax.experimental import pallas as pl


S_BLK = 2048


def _add_kernel(x_ref, emb_ref, out_ref):
    out_ref[...] = x_ref[...] + emb_ref[...]


def kernel(x, emb):
    batch, seq_len, d_model = x.shape
    n_s = seq_len // S_BLK
    return pl.pallas_call(
        _add_kernel,
        grid=(n_s, batch),
        in_specs=[
            pl.BlockSpec((1, S_BLK, d_model), lambda i_s, i_b: (i_b, i_s, 0)),
            pl.BlockSpec((S_BLK, d_model), lambda i_s, i_b: (i_s, 0)),
        ],
        out_specs=pl.BlockSpec((1, S_BLK, d_model), lambda i_s, i_b: (i_b, i_s, 0)),
        out_shape=jax.ShapeDtypeStruct(x.shape, x.dtype),
    )(x, emb)
